# Initial kernel scaffold; baseline (speedup 1.0000x reference)
import jax
import jax.numpy as jnp
from jax.experimental import pallas as pl


def kernel(x, weight):
    raise NotImplementedError("write your pallas kernel here")


# speedup vs baseline: 3.3481x; 3.3481x over previous
import functools

import jax
import jax.numpy as jnp
from jax import lax
from jax.experimental import pallas as pl
from jax.experimental.pallas import tpu as pltpu
from jax.experimental.pallas import tpu_sc as plsc

NBUF = 5
PREFETCH = 2


@functools.lru_cache(maxsize=None)
def _build(n_rows: int, vocab: int, dim: int):
    info = plsc.get_sparse_core_info()
    nw = info.num_cores * info.num_subcores
    chunk = 128
    rows_per_w = n_rows // nw
    nchunk = rows_per_w // chunk
    assert rows_per_w % chunk == 0 and nchunk % NBUF == 0

    mesh = plsc.VectorSubcoreMesh(core_axis_name="c", subcore_axis_name="s")

    @functools.partial(
        pl.kernel,
        mesh=mesh,
        out_type=jax.ShapeDtypeStruct((n_rows // chunk, chunk, dim), jnp.float32),
        scratch_types=(
            [pltpu.VMEM((nchunk, chunk), jnp.int32)]
            + [pltpu.VMEM((chunk, dim), jnp.float32) for _ in range(NBUF)]
            + [pltpu.SemaphoreType.DMA for _ in range(2 * NBUF)]
        ),
    )
    def gather_kernel(x_hbm, w_hbm, out_hbm, idx_v, *rest):
        bufs = rest[:NBUF]
        gsems = rest[NBUF : 2 * NBUF]
        ssems = rest[2 * NBUF :]
        wid = lax.axis_index("s") * info.num_cores + lax.axis_index("c")
        chunk0 = wid * nchunk

        pltpu.sync_copy(x_hbm.at[wid], idx_v)

        def start_gather(j, b):
            pltpu.async_copy(w_hbm.at[idx_v.at[j]], bufs[b], gsems[b])

        def wait_gather(j, b):
            pltpu.make_async_copy(w_hbm.at[idx_v.at[j]], bufs[b], gsems[b]).wait()

        def start_store(j, b):
            pltpu.async_copy(bufs[b], out_hbm.at[chunk0 + j], ssems[b])

        def wait_store(j, b):
            pltpu.make_async_copy(bufs[b], out_hbm.at[chunk0 + j], ssems[b]).wait()

        for b in range(PREFETCH):
            start_gather(b, b)

        @pl.loop(0, nchunk, step=NBUF, unroll=False)
        def _group(g):
            for b in range(NBUF):
                j = g + b
                t = j + PREFETCH
                tb = (b + PREFETCH) % NBUF

                @pl.when(t - NBUF >= 0)
                def _():
                    wait_store(t - NBUF, tb)

                @pl.when(t < nchunk)
                def _():
                    start_gather(t, tb)

                wait_gather(j, b)
                start_store(j, b)

        for j in range(nchunk - NBUF + PREFETCH, nchunk):
            wait_store(j, j % NBUF)

    return gather_kernel, nw, chunk, nchunk


def kernel(x, weight):
    b, s = x.shape
    vocab, dim = weight.shape
    n_rows = b * s
    gather_kernel, nw, chunk, nchunk = _build(n_rows, vocab, dim)
    x_blocked = x.reshape(nw, nchunk, chunk).astype(jnp.int32)
    out = gather_kernel(x_blocked, weight)
    return out.reshape(b, s, dim)
